# baseline (speedup 1.0000x reference)
import functools

import jax
import jax.numpy as jnp
from jax import lax
from jax.experimental import pallas as pl
from jax.experimental.pallas import tpu as pltpu
from jax.experimental.pallas import tpu_sc as plsc

EMBEDDING_DIM = 64
_NUM_CORES = 2
_NUM_SUBCORES = 16
_NW = _NUM_CORES * _NUM_SUBCORES

_NBUF = 8
_AHEAD = 4


def _build(batch: int, seq: int):
    assert batch % _NW == 0
    rows_per_w = batch // _NW
    sizes = (128, seq - 128)
    n_chunks = 2 * rows_per_w
    assert n_chunks % _NBUF == 0
    n_groups = n_chunks // _NBUF

    mesh = plsc.VectorSubcoreMesh(core_axis_name="c", subcore_axis_name="s")

    @functools.partial(
        pl.kernel,
        mesh=mesh,
        compiler_params=pltpu.CompilerParams(use_tc_tiling_on_sc=False),
        out_type=jax.ShapeDtypeStruct((batch, seq, EMBEDDING_DIM), jnp.float32),
        scratch_types=[
            pltpu.VMEM((rows_per_w, seq), jnp.int32),
            pltpu.VMEM((_NBUF, 128, EMBEDDING_DIM), jnp.float32),
            pltpu.SemaphoreType.DMA((_NBUF,)),
            pltpu.SemaphoreType.DMA((_NBUF,)),
        ],
    )
    def emb(idx_hbm, table_hbm, out_hbm, idx_v, rows_v, gsem, ssem):
        wid = lax.axis_index("s") * _NUM_CORES + lax.axis_index("c")
        base_row = wid * rows_per_w

        pltpu.sync_copy(idx_hbm.at[pl.ds(base_row, rows_per_w), :], idx_v)

        def issue_gather(j, b):
            half = b % 2
            sz = sizes[half]
            pltpu.async_copy(
                table_hbm.at[idx_v.at[j // 2, pl.ds(128 * half, sz)]],
                rows_v.at[b, pl.ds(0, sz)],
                gsem.at[b],
            )

        def wait_gather(j, b):
            half = b % 2
            sz = sizes[half]
            pltpu.make_async_copy(
                table_hbm.at[idx_v.at[j // 2, pl.ds(128 * half, sz)]],
                rows_v.at[b, pl.ds(0, sz)],
                gsem.at[b],
            ).wait()

        def issue_store(j, b):
            half = b % 2
            sz = sizes[half]
            pltpu.async_copy(
                rows_v.at[b, pl.ds(0, sz)],
                out_hbm.at[base_row + j // 2, pl.ds(128 * half, sz), :],
                ssem.at[b],
            )

        def wait_store(b):
            sz = sizes[b % 2]
            pltpu.make_async_copy(
                rows_v.at[b, pl.ds(0, sz)],
                out_hbm.at[base_row, pl.ds(128 * (b % 2), sz), :],
                ssem.at[b],
            ).wait()

        for b in range(_AHEAD):
            issue_gather(b, b)

        def group_body(g, carry):
            for b in range(_NBUF):
                j = g * _NBUF + b
                wait_gather(j, b)
                issue_store(j, b)
                c = (b + _AHEAD) % _NBUF

                @pl.when(jnp.logical_and(j + _AHEAD >= _NBUF,
                                         j + _AHEAD < n_chunks))
                def _():
                    wait_store(c)

                @pl.when(j + _AHEAD < n_chunks)
                def _():
                    issue_gather(j + _AHEAD, c)

            return carry

        lax.fori_loop(0, n_groups, group_body, 0)

        for b in range(_NBUF):
            wait_store(b)

    return emb


def kernel(input, weight):
    batch, seq = input.shape
    return _build(batch, seq)(input.astype(jnp.int32), weight)
